# baseline (speedup 1.0000x reference)
import jax
import jax.numpy as jnp
from jax.experimental import pallas as pl
from jax.experimental.pallas import tpu as pltpu

K = 100000
DIM = 128
B = 16384

_SIZES = (32768, 32768, 16384, 16384, 1696)
_OFFS = tuple(sum(_SIZES[:i]) for i in range(len(_SIZES)))
_NCH = len(_SIZES)
_NBUF = 3
_BUF_ROWS = max(_SIZES)


def _in_copies(i, bank_ref, value_ref, buf, sem):
    r0, r1 = _OFFS[i], _OFFS[i] + _SIZES[i]
    copies = []
    if r0 < B:
        n = min(r1, B) - r0
        copies.append(pltpu.make_async_copy(
            value_ref.at[pl.ds(r0, n)], buf.at[pl.ds(0, n)], sem))
    if r1 > B:
        s = max(r0, B)
        n = r1 - s
        copies.append(pltpu.make_async_copy(
            bank_ref.at[pl.ds(s, n)], buf.at[pl.ds(s - r0, n)], sem))
    return copies


def _push_body(bank_ref, value_ref, out_ref, *scratch):
    bufs, sin, sout = scratch[:_NBUF], scratch[_NBUF:2 * _NBUF], scratch[2 * _NBUF:]
    ins = [_in_copies(i, bank_ref, value_ref, bufs[i % _NBUF], sin[i % _NBUF])
           for i in range(_NCH)]
    outs = [pltpu.make_async_copy(
        bufs[i % _NBUF].at[pl.ds(0, _SIZES[i])],
        out_ref.at[pl.ds(_OFFS[i], _SIZES[i])], sout[i % _NBUF])
        for i in range(_NCH)]
    for i in range(min(_NBUF, _NCH)):
        for c in ins[i]:
            c.start()
    for i in range(_NCH):
        for c in ins[i]:
            c.wait()
        outs[i].start()
        if i + _NBUF < _NCH:
            outs[i].wait()
            for c in ins[i + _NBUF]:
                c.start()
    for i in range(max(0, _NCH - _NBUF), _NCH):
        outs[i].wait()


@jax.jit
def kernel(bank, value):
    return pl.pallas_call(
        _push_body,
        out_shape=jax.ShapeDtypeStruct((K, DIM), jnp.float32),
        in_specs=[
            pl.BlockSpec(memory_space=pl.ANY),
            pl.BlockSpec(memory_space=pl.ANY),
        ],
        out_specs=pl.BlockSpec(memory_space=pl.ANY),
        scratch_shapes=(
            [pltpu.VMEM((_BUF_ROWS, DIM), jnp.float32)] * _NBUF
            + [pltpu.SemaphoreType.DMA] * (2 * _NBUF)
        ),
    )(bank, value)


# speedup vs baseline: 1.0163x; 1.0163x over previous
import jax
import jax.numpy as jnp
from jax.experimental import pallas as pl
from jax.experimental.pallas import tpu as pltpu

K = 100000
DIM = 128
B = 16384

_CH = 32768
_NCH = (K + _CH - 1) // _CH


def _rows(i):
    return min(_CH, K - i * _CH)


def _in_copies(i, bank_ref, value_ref, buf, sem):
    r0, r1 = i * _CH, i * _CH + _rows(i)
    copies = []
    if r0 < B:
        n = min(r1, B) - r0
        copies.append(pltpu.make_async_copy(
            value_ref.at[pl.ds(r0, n)], buf.at[pl.ds(0, n)], sem))
    if r1 > B:
        s = max(r0, B)
        n = r1 - s
        copies.append(pltpu.make_async_copy(
            bank_ref.at[pl.ds(s, n)], buf.at[pl.ds(s - r0, n)], sem))
    return copies


def _push_body(bank_ref, value_ref, out_ref, *scratch):
    bufs, sin, sout = scratch[:_NCH], scratch[_NCH:2 * _NCH], scratch[2 * _NCH:]
    ins = [_in_copies(i, bank_ref, value_ref, bufs[i], sin[i])
           for i in range(_NCH)]
    outs = [pltpu.make_async_copy(bufs[i], out_ref.at[pl.ds(i * _CH, _rows(i))],
                                  sout[i])
            for i in range(_NCH)]
    for cs in ins:
        for c in cs:
            c.start()
    for i in range(_NCH):
        for c in ins[i]:
            c.wait()
        outs[i].start()
    for c in outs:
        c.wait()


@jax.jit
def kernel(bank, value):
    return pl.pallas_call(
        _push_body,
        out_shape=jax.ShapeDtypeStruct((K, DIM), jnp.float32),
        in_specs=[
            pl.BlockSpec(memory_space=pl.ANY),
            pl.BlockSpec(memory_space=pl.ANY),
        ],
        out_specs=pl.BlockSpec(memory_space=pl.ANY),
        scratch_shapes=(
            [pltpu.VMEM((_rows(i), DIM), jnp.float32) for i in range(_NCH)]
            + [pltpu.SemaphoreType.DMA] * (2 * _NCH)
        ),
    )(bank, value)


# speedup vs baseline: 1.0328x; 1.0162x over previous
import jax
import jax.numpy as jnp
from jax.experimental import pallas as pl
from jax.experimental.pallas import tpu as pltpu

K = 100000
DIM = 128
B = 16384

_CH = 32768
_NCH = (K + _CH - 1) // _CH
_NBUF = 3


def _rows(i):
    return min(_CH, K - i * _CH)


def _in_copies(i, bank_ref, value_ref, buf, sem):
    r0, r1 = i * _CH, i * _CH + _rows(i)
    copies = []
    if r0 < B:
        n = min(r1, B) - r0
        copies.append(pltpu.make_async_copy(
            value_ref.at[pl.ds(r0, n)], buf.at[pl.ds(0, n)], sem))
    if r1 > B:
        s = max(r0, B)
        n = r1 - s
        copies.append(pltpu.make_async_copy(
            bank_ref.at[pl.ds(s, n)], buf.at[pl.ds(s - r0, n)], sem))
    return copies


def _push_body(bank_ref, value_ref, out_ref, *scratch):
    bufs, sin, sout = scratch[:_NBUF], scratch[_NBUF:2 * _NBUF], scratch[2 * _NBUF:]
    ins = [_in_copies(i, bank_ref, value_ref, bufs[i % _NBUF], sin[i % _NBUF])
           for i in range(_NCH)]
    outs = [pltpu.make_async_copy(
        bufs[i % _NBUF].at[pl.ds(0, _rows(i))],
        out_ref.at[pl.ds(i * _CH, _rows(i))], sout[i % _NBUF])
        for i in range(_NCH)]
    for i in range(_NBUF):
        if i < _NCH:
            for c in ins[i]:
                c.start()
    for i in range(_NCH):
        for c in ins[i]:
            c.wait()
        outs[i].start()
        if i + _NBUF < _NCH:
            outs[i].wait()
            for c in ins[i + _NBUF]:
                c.start()
    for i in range(max(0, _NCH - _NBUF), _NCH):
        outs[i].wait()


@jax.jit
def kernel(bank, value):
    return pl.pallas_call(
        _push_body,
        out_shape=jax.ShapeDtypeStruct((K, DIM), jnp.float32),
        in_specs=[
            pl.BlockSpec(memory_space=pl.ANY),
            pl.BlockSpec(memory_space=pl.ANY),
        ],
        out_specs=pl.BlockSpec(memory_space=pl.ANY),
        scratch_shapes=(
            [pltpu.VMEM((_CH, DIM), jnp.float32)] * _NBUF
            + [pltpu.SemaphoreType.DMA] * (2 * _NBUF)
        ),
    )(bank, value)
